# baseline (speedup 1.0000x reference)
import functools

import jax
import jax.numpy as jnp
from jax import lax
from jax.experimental import pallas as pl
from jax.experimental.pallas import tpu as pltpu
from jax.experimental.pallas import tpu_sc as plsc

NC = 2
NS = 16
NW = NC * NS
L = 16

D = 64
CH = 64
NG = D // L


def _sc_score(B, n_chunks):
    mesh = plsc.VectorSubcoreMesh(
        core_axis_name="c", subcore_axis_name="s", num_cores=NC, num_subcores=NS
    )
    q_per_w = B // NW
    assert q_per_w == n_chunks * CH and n_chunks % 2 == 0

    idx_buf = lambda: pltpu.VMEM((CH,), jnp.int32)
    ebuf = lambda: pltpu.VMEM((CH, 2 * D), jnp.float32)

    @functools.partial(
        pl.kernel,
        out_type=jax.ShapeDtypeStruct((B,), jnp.float32),
        mesh=mesh,
        compiler_params=pltpu.CompilerParams(needs_layout_passes=False),
        scratch_types=dict(
            idx=[[idx_buf() for _ in range(4)] for _ in range(2)],
            sbuf=[ebuf() for _ in range(2)],
            obuf=[ebuf() for _ in range(2)],
            rbuf=[pltpu.VMEM((CH, 6 * D), jnp.float32) for _ in range(2)],
            tbuf=[ebuf() for _ in range(2)],
            part=pltpu.VMEM((L, L), jnp.float32),
            res=pltpu.VMEM((CH,), jnp.float32),
            sem=[pltpu.SemaphoreType.DMA for _ in range(2)],
        ),
    )
    def score(idx_h, Ecat, Rcat, Tcat, out_h, *, idx, sbuf, obuf, rbuf, tbuf,
              part, res, sem):
        wid = lax.axis_index("s") * NC + lax.axis_index("c")
        base = wid * q_per_w

        def fetch(ch, slot):
            cbase = base + ch * CH
            for k in range(4):
                pltpu.sync_copy(idx_h.at[pl.ds(k * B + cbase, CH)], idx[slot][k])
            pltpu.async_copy(Ecat.at[idx[slot][0]], sbuf[slot], sem[slot])
            pltpu.async_copy(Rcat.at[idx[slot][1]], rbuf[slot], sem[slot])
            pltpu.async_copy(Ecat.at[idx[slot][2]], obuf[slot], sem[slot])
            pltpu.async_copy(Tcat.at[idx[slot][3]], tbuf[slot], sem[slot])

        def drain(slot):
            pltpu.make_async_copy(Ecat.at[idx[slot][0]], sbuf[slot], sem[slot]).wait()
            pltpu.make_async_copy(Rcat.at[idx[slot][1]], rbuf[slot], sem[slot]).wait()
            pltpu.make_async_copy(Ecat.at[idx[slot][2]], obuf[slot], sem[slot]).wait()
            pltpu.make_async_copy(Tcat.at[idx[slot][3]], tbuf[slot], sem[slot]).wait()

        def compute(slot):
            sb, ob, rb, tb = sbuf[slot], obuf[slot], rbuf[slot], tbuf[slot]

            def gbody(qg, carry):
                for j in range(L):
                    q = qg * L + j
                    acc = jnp.zeros((L,), jnp.float32)
                    for g in range(NG):
                        o0 = g * L
                        sr = sb[q, pl.ds(o0, L)]
                        si = sb[q, pl.ds(D + o0, L)]
                        orv = ob[q, pl.ds(o0, L)]
                        oi = ob[q, pl.ds(D + o0, L)]
                        rr = rb[q, pl.ds(o0, L)]
                        ri = rb[q, pl.ds(D + o0, L)]
                        rsr = rb[q, pl.ds(2 * D + o0, L)]
                        rsi = rb[q, pl.ds(3 * D + o0, L)]
                        ror = rb[q, pl.ds(4 * D + o0, L)]
                        roi = rb[q, pl.ds(5 * D + o0, L)]
                        tr = tb[q, pl.ds(o0, L)]
                        ti = tb[q, pl.ds(D + o0, L)]
                        a = sr * rr - si * ri + tr * ror - ti * roi
                        b = sr * ri + si * rr + tr * roi + ti * ror
                        c = sr * rsr - si * rsi
                        d = sr * rsi + si * rsr
                        acc = acc + (a * orv + b * oi + c * tr + d * ti)
                    part[j] = plsc.cumsum(acc)
                rows = lax.iota(jnp.int32, L)
                cols = jnp.full((L,), L - 1, jnp.int32)
                res[pl.ds(qg * L, L)] = plsc.load_gather(part, [rows, cols])
                return carry

            lax.fori_loop(0, CH // L, gbody, 0)

        npairs = n_chunks // 2
        fetch(0, 0)

        def pair_body(p, carry):
            c0 = 2 * p
            drain(0)
            fetch(c0 + 1, 1)
            compute(0)
            pltpu.sync_copy(res, out_h.at[pl.ds(base + c0 * CH, CH)])
            drain(1)

            @pl.when(p + 1 < npairs)
            def _():
                fetch(c0 + 2, 0)

            compute(1)
            pltpu.sync_copy(res, out_h.at[pl.ds(base + (c0 + 1) * CH, CH)])
            return carry

        lax.fori_loop(0, npairs, pair_body, 0)

    return score


def _fuse_entity_tables(e_re_t, e_im_t):
    V = e_re_t.shape[1]
    C = 16384
    def body(re_ref, im_ref, out_ref):
        eye = (
            lax.broadcasted_iota(jnp.int32, (D, D), 0)
            == lax.broadcasted_iota(jnp.int32, (D, D), 1)
        ).astype(jnp.float32)
        dims = (((0,), (0,)), ((), ()))
        out_ref[:, 0:D] = lax.dot_general(
            re_ref[...], eye, dims,
            preferred_element_type=jnp.float32, precision=lax.Precision.DEFAULT)
        out_ref[:, D:2 * D] = lax.dot_general(
            im_ref[...], eye, dims,
            preferred_element_type=jnp.float32, precision=lax.Precision.DEFAULT)
    return pl.pallas_call(
        body,
        grid=(pl.cdiv(V, C),),
        in_specs=[
            pl.BlockSpec((D, C), lambda i: (0, i)),
            pl.BlockSpec((D, C), lambda i: (0, i)),
        ],
        out_specs=pl.BlockSpec((C, 2 * D), lambda i: (i, 0)),
        out_shape=jax.ShapeDtypeStruct((V, 2 * D), jnp.float32),
    )(e_re_t, e_im_t)


def kernel(s, r, o, t, E_im, E_re, R_im, R_re, Rs_im, Rs_re, Ro_im, Ro_re,
           Ts_im, Ts_re, To_im, To_re):
    del To_im, To_re
    B = s.shape[0]
    idx_h = jnp.concatenate(
        [s.reshape(B), r.reshape(B), o.reshape(B), t[:, 0, 0]]
    ).astype(jnp.int32)
    Ecat = _fuse_entity_tables(E_re.T, E_im.T)
    Rcat = jnp.concatenate([R_re, R_im, Rs_re, Rs_im, Ro_re, Ro_im], axis=1)
    Tcat = jnp.concatenate([Ts_re, Ts_im], axis=1)
    score = _sc_score(B, B // (NW * CH))
    out = score(idx_h, Ecat, Rcat, Tcat)
    return out.reshape(B, 1)


# speedup vs baseline: 1.0111x; 1.0111x over previous
import functools

import jax
import jax.numpy as jnp
from jax import lax
from jax.experimental import pallas as pl
from jax.experimental.pallas import tpu as pltpu
from jax.experimental.pallas import tpu_sc as plsc

NC = 2
NS = 16
NW = NC * NS
L = 16

D = 64
CH = 64
NG = D // L


def _sc_score(B, n_chunks):
    mesh = plsc.VectorSubcoreMesh(
        core_axis_name="c", subcore_axis_name="s", num_cores=NC, num_subcores=NS
    )
    q_per_w = B // NW
    assert q_per_w == n_chunks * CH and n_chunks % 2 == 0

    idx_buf = lambda: pltpu.VMEM((CH,), jnp.int32)
    ebuf = lambda: pltpu.VMEM((CH, 2 * D), jnp.float32)

    @functools.partial(
        pl.kernel,
        out_type=jax.ShapeDtypeStruct((B,), jnp.float32),
        mesh=mesh,
        compiler_params=pltpu.CompilerParams(needs_layout_passes=False),
        scratch_types=dict(
            idx=[[idx_buf() for _ in range(4)] for _ in range(2)],
            sbuf=[ebuf() for _ in range(2)],
            obuf=[ebuf() for _ in range(2)],
            rbuf=[pltpu.VMEM((CH, 6 * D), jnp.float32) for _ in range(2)],
            tbuf=[ebuf() for _ in range(2)],
            part=pltpu.VMEM((L, L), jnp.float32),
            res=pltpu.VMEM((CH,), jnp.float32),
            sem=[pltpu.SemaphoreType.DMA for _ in range(2)],
        ),
    )
    def score(idx_h, Ecat, Rcat, Tcat, out_h, *, idx, sbuf, obuf, rbuf, tbuf,
              part, res, sem):
        wid = lax.axis_index("s") * NC + lax.axis_index("c")
        base = wid * q_per_w

        def fetch(ch, slot):
            cbase = base + ch * CH
            for k in range(4):
                pltpu.sync_copy(idx_h.at[pl.ds(k * B + cbase, CH)], idx[slot][k])
            pltpu.async_copy(Ecat.at[idx[slot][0]], sbuf[slot], sem[slot])
            pltpu.async_copy(Rcat.at[idx[slot][1]], rbuf[slot], sem[slot])
            pltpu.async_copy(Ecat.at[idx[slot][2]], obuf[slot], sem[slot])
            pltpu.async_copy(Tcat.at[idx[slot][3]], tbuf[slot], sem[slot])

        def drain(slot):
            pltpu.make_async_copy(Ecat.at[idx[slot][0]], sbuf[slot], sem[slot]).wait()
            pltpu.make_async_copy(Rcat.at[idx[slot][1]], rbuf[slot], sem[slot]).wait()
            pltpu.make_async_copy(Ecat.at[idx[slot][2]], obuf[slot], sem[slot]).wait()
            pltpu.make_async_copy(Tcat.at[idx[slot][3]], tbuf[slot], sem[slot]).wait()

        def compute(slot):
            sb, ob, rb, tb = sbuf[slot], obuf[slot], rbuf[slot], tbuf[slot]

            def gbody(qg, carry):
                for j in range(L):
                    q = qg * L + j
                    acc = jnp.zeros((L,), jnp.float32)
                    for g in range(NG):
                        o0 = g * L
                        sr = sb[q, pl.ds(o0, L)]
                        si = sb[q, pl.ds(D + o0, L)]
                        orv = ob[q, pl.ds(o0, L)]
                        oi = ob[q, pl.ds(D + o0, L)]
                        rr = rb[q, pl.ds(o0, L)]
                        ri = rb[q, pl.ds(D + o0, L)]
                        rsr = rb[q, pl.ds(2 * D + o0, L)]
                        rsi = rb[q, pl.ds(3 * D + o0, L)]
                        ror = rb[q, pl.ds(4 * D + o0, L)]
                        roi = rb[q, pl.ds(5 * D + o0, L)]
                        tr = tb[q, pl.ds(o0, L)]
                        ti = tb[q, pl.ds(D + o0, L)]
                        a = sr * rr - si * ri + tr * ror - ti * roi
                        b = sr * ri + si * rr + tr * roi + ti * ror
                        c = sr * rsr - si * rsi
                        d = sr * rsi + si * rsr
                        acc = acc + (a * orv + b * oi + c * tr + d * ti)
                    part[j] = plsc.cumsum(acc)
                rows = lax.iota(jnp.int32, L)
                cols = jnp.full((L,), L - 1, jnp.int32)
                res[pl.ds(qg * L, L)] = plsc.load_gather(part, [rows, cols])
                return carry

            lax.fori_loop(0, CH // L, gbody, 0)

        npairs = n_chunks // 2
        fetch(0, 0)

        def pair_body(p, carry):
            c0 = 2 * p
            drain(0)
            fetch(c0 + 1, 1)
            compute(0)
            pltpu.sync_copy(res, out_h.at[pl.ds(base + c0 * CH, CH)])
            drain(1)

            @pl.when(p + 1 < npairs)
            def _():
                fetch(c0 + 2, 0)

            compute(1)
            pltpu.sync_copy(res, out_h.at[pl.ds(base + (c0 + 1) * CH, CH)])
            return carry

        lax.fori_loop(0, npairs, pair_body, 0)

    return score


def _fuse_entity_tables(e_re_t, e_im_t):
    V = e_re_t.shape[1]
    C = 8192
    def body(re_ref, im_ref, out_ref):
        eye = (
            lax.broadcasted_iota(jnp.int32, (D, D), 0)
            == lax.broadcasted_iota(jnp.int32, (D, D), 1)
        ).astype(jnp.float32)
        dims = (((0,), (0,)), ((), ()))
        out_ref[:, 0:D] = lax.dot_general(
            re_ref[...], eye, dims,
            preferred_element_type=jnp.float32, precision=lax.Precision.DEFAULT)
        out_ref[:, D:2 * D] = lax.dot_general(
            im_ref[...], eye, dims,
            preferred_element_type=jnp.float32, precision=lax.Precision.DEFAULT)
    return pl.pallas_call(
        body,
        grid=(pl.cdiv(V, C),),
        in_specs=[
            pl.BlockSpec((D, C), lambda i: (0, i)),
            pl.BlockSpec((D, C), lambda i: (0, i)),
        ],
        out_specs=pl.BlockSpec((C, 2 * D), lambda i: (i, 0)),
        out_shape=jax.ShapeDtypeStruct((V, 2 * D), jnp.float32),
    )(e_re_t, e_im_t)


def kernel(s, r, o, t, E_im, E_re, R_im, R_re, Rs_im, Rs_re, Ro_im, Ro_re,
           Ts_im, Ts_re, To_im, To_re):
    del To_im, To_re
    B = s.shape[0]
    idx_h = jnp.concatenate(
        [s.reshape(B), r.reshape(B), o.reshape(B), t[:, 0, 0]]
    ).astype(jnp.int32)
    Ecat = _fuse_entity_tables(E_re.T, E_im.T)
    Rcat = jnp.concatenate([R_re, R_im, Rs_re, Rs_im, Ro_re, Ro_im], axis=1)
    Tcat = jnp.concatenate([Ts_re, Ts_im], axis=1)
    score = _sc_score(B, B // (NW * CH))
    out = score(idx_h, Ecat, Rcat, Tcat)
    return out.reshape(B, 1)


# speedup vs baseline: 1.0785x; 1.0666x over previous
import functools

import jax
import jax.numpy as jnp
from jax import lax
from jax.experimental import pallas as pl
from jax.experimental.pallas import tpu as pltpu
from jax.experimental.pallas import tpu_sc as plsc

NC = 2
NS = 16
NW = NC * NS
L = 16

D = 64
CH = 64
NG = D // L


def _sc_score(B, n_chunks):
    mesh = plsc.VectorSubcoreMesh(
        core_axis_name="c", subcore_axis_name="s", num_cores=NC, num_subcores=NS
    )
    q_per_w = B // NW
    assert q_per_w == n_chunks * CH and n_chunks % 2 == 0

    idx_buf = lambda: pltpu.VMEM((CH,), jnp.int32)
    ebuf = lambda: pltpu.VMEM((CH, 2 * D), jnp.float32)

    @functools.partial(
        pl.kernel,
        out_type=jax.ShapeDtypeStruct((B,), jnp.float32),
        mesh=mesh,
        compiler_params=pltpu.CompilerParams(needs_layout_passes=False),
        scratch_types=dict(
            soidx=pltpu.VMEM((2 * CH * n_chunks,), jnp.int32),
            ridx=pltpu.VMEM((CH * n_chunks,), jnp.int32),
            tidx=pltpu.VMEM((CH * n_chunks,), jnp.int32),
            sobuf=[pltpu.VMEM((2 * CH, 2 * D), jnp.float32) for _ in range(2)],
            rbuf=[pltpu.VMEM((CH, 6 * D), jnp.float32) for _ in range(2)],
            tbuf=[pltpu.VMEM((CH, 2 * D), jnp.float32) for _ in range(2)],
            part=pltpu.VMEM((L, L), jnp.float32),
            res=pltpu.VMEM((CH,), jnp.float32),
            sem=[pltpu.SemaphoreType.DMA for _ in range(2)],
        ),
    )
    def score(idx_h, Ecat, Rcat, Tcat, out_h, *, soidx, ridx, tidx, sobuf,
              rbuf, tbuf, part, res, sem):
        wid = lax.axis_index("s") * NC + lax.axis_index("c")
        base = wid * q_per_w
        pltpu.sync_copy(idx_h.at[pl.ds(2 * base, 2 * q_per_w)], soidx)
        pltpu.sync_copy(idx_h.at[pl.ds(2 * B + base, q_per_w)], ridx)
        pltpu.sync_copy(idx_h.at[pl.ds(3 * B + base, q_per_w)], tidx)

        def fetch(ch, slot):
            pltpu.async_copy(
                Ecat.at[soidx.at[pl.ds(2 * CH * ch, 2 * CH)]], sobuf[slot],
                sem[slot])
            pltpu.async_copy(
                Rcat.at[ridx.at[pl.ds(CH * ch, CH)]], rbuf[slot], sem[slot])
            pltpu.async_copy(
                Tcat.at[tidx.at[pl.ds(CH * ch, CH)]], tbuf[slot], sem[slot])

        def drain(slot):
            pltpu.make_async_copy(
                Ecat.at[soidx.at[pl.ds(0, 2 * CH)]], sobuf[slot],
                sem[slot]).wait()
            pltpu.make_async_copy(
                Rcat.at[ridx.at[pl.ds(0, CH)]], rbuf[slot], sem[slot]).wait()
            pltpu.make_async_copy(
                Tcat.at[tidx.at[pl.ds(0, CH)]], tbuf[slot], sem[slot]).wait()

        def compute(slot):
            so, rb, tb = sobuf[slot], rbuf[slot], tbuf[slot]

            def gbody(qg, carry):
                for j in range(L):
                    q = qg * L + j
                    acc = jnp.zeros((L,), jnp.float32)
                    for g in range(NG):
                        o0 = g * L
                        sr = so[q, pl.ds(o0, L)]
                        si = so[q, pl.ds(D + o0, L)]
                        orv = so[CH + q, pl.ds(o0, L)]
                        oi = so[CH + q, pl.ds(D + o0, L)]
                        rr = rb[q, pl.ds(o0, L)]
                        ri = rb[q, pl.ds(D + o0, L)]
                        rsr = rb[q, pl.ds(2 * D + o0, L)]
                        rsi = rb[q, pl.ds(3 * D + o0, L)]
                        ror = rb[q, pl.ds(4 * D + o0, L)]
                        roi = rb[q, pl.ds(5 * D + o0, L)]
                        tr = tb[q, pl.ds(o0, L)]
                        ti = tb[q, pl.ds(D + o0, L)]
                        a = sr * rr - si * ri + tr * ror - ti * roi
                        b = sr * ri + si * rr + tr * roi + ti * ror
                        c = sr * rsr - si * rsi
                        d = sr * rsi + si * rsr
                        acc = acc + (a * orv + b * oi + c * tr + d * ti)
                    part[j] = plsc.cumsum(acc)
                rows = lax.iota(jnp.int32, L)
                cols = jnp.full((L,), L - 1, jnp.int32)
                res[pl.ds(qg * L, L)] = plsc.load_gather(part, [rows, cols])
                return carry

            lax.fori_loop(0, CH // L, gbody, 0)

        npairs = n_chunks // 2
        fetch(0, 0)

        def pair_body(p, carry):
            c0 = 2 * p
            drain(0)
            fetch(c0 + 1, 1)
            compute(0)
            pltpu.sync_copy(res, out_h.at[pl.ds(base + c0 * CH, CH)])
            drain(1)

            @pl.when(p + 1 < npairs)
            def _():
                fetch(c0 + 2, 0)

            compute(1)
            pltpu.sync_copy(res, out_h.at[pl.ds(base + (c0 + 1) * CH, CH)])
            return carry

        lax.fori_loop(0, npairs, pair_body, 0)

    return score


def _fuse_entity_tables(e_re_t, e_im_t):
    V = e_re_t.shape[1]
    C = 8192
    def body(re_ref, im_ref, out_ref):
        eye = (
            lax.broadcasted_iota(jnp.int32, (D, D), 0)
            == lax.broadcasted_iota(jnp.int32, (D, D), 1)
        ).astype(jnp.float32)
        dims = (((0,), (0,)), ((), ()))
        out_ref[:, 0:D] = lax.dot_general(
            re_ref[...], eye, dims,
            preferred_element_type=jnp.float32, precision=lax.Precision.DEFAULT)
        out_ref[:, D:2 * D] = lax.dot_general(
            im_ref[...], eye, dims,
            preferred_element_type=jnp.float32, precision=lax.Precision.DEFAULT)
    return pl.pallas_call(
        body,
        grid=(pl.cdiv(V, C),),
        in_specs=[
            pl.BlockSpec((D, C), lambda i: (0, i)),
            pl.BlockSpec((D, C), lambda i: (0, i)),
        ],
        out_specs=pl.BlockSpec((C, 2 * D), lambda i: (i, 0)),
        out_shape=jax.ShapeDtypeStruct((V, 2 * D), jnp.float32),
    )(e_re_t, e_im_t)


def kernel(s, r, o, t, E_im, E_re, R_im, R_re, Rs_im, Rs_re, Ro_im, Ro_re,
           Ts_im, Ts_re, To_im, To_re):
    del To_im, To_re
    B = s.shape[0]
    so_idx = jnp.stack(
        [s.reshape(-1, CH), o.reshape(-1, CH)], axis=1
    ).reshape(2 * B)
    idx_h = jnp.concatenate(
        [so_idx, r.reshape(B), t[:, 0, 0]]
    ).astype(jnp.int32)
    Ecat = _fuse_entity_tables(E_re.T, E_im.T)
    Rcat = jnp.concatenate([R_re, R_im, Rs_re, Rs_im, Ro_re, Ro_im], axis=1)
    Tcat = jnp.concatenate([Ts_re, Ts_im], axis=1)
    score = _sc_score(B, B // (NW * CH))
    out = score(idx_h, Ecat, Rcat, Tcat)
    return out.reshape(B, 1)
